# baseline (speedup 1.0000x reference)
import math

import jax
import jax.numpy as jnp
from jax import lax
from jax.experimental import pallas as pl
from jax.experimental.pallas import tpu as pltpu
from jax.experimental.pallas import tpu_sc as plsc

N = 50000
E = 800000
F_IN = 50
H = 128
C_OUT = 121
ALPHA = 0.1
THETA = 0.5

NC = 2
NS = 16
G = 4
GC = H // G
K = 250
EK = E // K
IRT = EK // NS
BRK = 8
NBLK = IRT // BRK
ZST = 3200
ZREM = N - 15 * ZST


def _sc_pass(table, out_g, src2_hbm, dst2_hbm, zeros_hbm,
             dst_blk, src_blk, rows0, rows1, agg,
             gsem0, gsem1, ssem0, ssem1, tid):
    @pl.when(tid < 15)
    def _zero_main():
        pltpu.sync_copy(zeros_hbm, agg.at[pl.ds(tid * ZST, ZST)])

    @pl.when(tid == 15)
    def _zero_rem():
        pltpu.sync_copy(zeros_hbm.at[pl.ds(0, ZREM)],
                        agg.at[pl.ds(15 * ZST, ZREM)])

    plsc.subcore_barrier()

    def blk_body(b, _):
        row_off = tid * IRT + b * BRK
        pltpu.sync_copy(src2_hbm.at[pl.ds(row_off, BRK)], src_blk)
        pltpu.sync_copy(dst2_hbm.at[pl.ds(row_off, BRK)], dst_blk)

        def gather(bb, rows, sem):
            pltpu.async_copy(table.at[src_blk.at[bb]], rows, sem)

        def gwait(rows, sem):
            pltpu.make_async_copy(table.at[src_blk.at[0]], rows, sem).wait()

        def scat(bb, rows, sem):
            pltpu.async_copy(rows, agg.at[dst_blk.at[bb]], sem, add=True)

        def swait(rows, sem):
            pltpu.make_async_copy(rows, agg.at[dst_blk.at[0]], sem).wait()

        gather(0, rows0, gsem0)
        gwait(rows0, gsem0)
        scat(0, rows0, ssem0)
        gather(1, rows1, gsem1)
        gwait(rows1, gsem1)
        scat(1, rows1, ssem1)
        swait(rows0, ssem0)
        gather(2, rows0, gsem0)

        def pair(i, _):
            b0 = 2 * i
            gwait(rows0, gsem0)
            scat(b0, rows0, ssem0)
            swait(rows1, ssem1)
            gather(b0 + 1, rows1, gsem1)
            gwait(rows1, gsem1)
            scat(b0 + 1, rows1, ssem1)
            swait(rows0, ssem0)
            gather(b0 + 2, rows0, gsem0)
            return 0

        lax.fori_loop(1, (BRK - 2) // 2, pair, 0)
        gwait(rows0, gsem0)
        scat(BRK - 2, rows0, ssem0)
        swait(rows1, ssem1)
        gather(BRK - 1, rows1, gsem1)
        gwait(rows1, gsem1)
        scat(BRK - 1, rows1, ssem1)
        swait(rows0, ssem0)
        swait(rows1, ssem1)
        return 0

    lax.fori_loop(0, NBLK, blk_body, 0)
    plsc.subcore_barrier()

    @pl.when(tid < 15)
    def _out_main():
        pltpu.sync_copy(agg.at[pl.ds(tid * ZST, ZST)],
                        out_g.at[pl.ds(tid * ZST, ZST)])

    @pl.when(tid == 15)
    def _out_rem():
        pltpu.sync_copy(agg.at[pl.ds(15 * ZST, ZREM)],
                        out_g.at[pl.ds(15 * ZST, ZREM)])

    plsc.subcore_barrier()


def _sc_scatter_body(h4_hbm, src2_hbm, dst2_hbm, zeros_hbm, out_hbm,
                     dst_blk, src_blk, rows0, rows1, agg,
                     gsem0, gsem1, ssem0, ssem1):
    core = lax.axis_index("c")
    tid = lax.axis_index("s")

    for c in range(NC):
        @pl.when(core == c)
        def _core_work(c=c):
            for gi in range(G // NC):
                g = c * (G // NC) + gi
                _sc_pass(h4_hbm.at[g], out_hbm.at[g], src2_hbm, dst2_hbm,
                         zeros_hbm, dst_blk, src_blk, rows0, rows1, agg,
                         gsem0, gsem1, ssem0, ssem1, tid)


def _sc_scatter(h4, src2, dst2, zeros32):
    mesh = plsc.VectorSubcoreMesh(core_axis_name="c", subcore_axis_name="s",
                                  num_cores=NC, num_subcores=NS)
    return pl.kernel(
        _sc_scatter_body,
        out_type=jax.ShapeDtypeStruct((G, N, GC), jnp.float32),
        mesh=mesh,
        scratch_types=[
            pltpu.VMEM((BRK, K), jnp.int32),
            pltpu.VMEM((BRK, K), jnp.int32),
            pltpu.VMEM((K, GC), jnp.float32),
            pltpu.VMEM((K, GC), jnp.float32),
            pltpu.VMEM_SHARED((N, GC), jnp.float32),
            pltpu.SemaphoreType.DMA,
            pltpu.SemaphoreType.DMA,
            pltpu.SemaphoreType.DMA,
            pltpu.SemaphoreType.DMA,
        ],
        compiler_params=pltpu.CompilerParams(use_tc_tiling_on_sc=False),
    )(h4, src2, dst2, zeros32)


R = 1000


def _split4(x):
    return [x[:, c * GC:(c + 1) * GC] for c in range(G)]


def _cat4(a4):
    return jnp.concatenate([a4[c] for c in range(G)], axis=-1)


def _lin0_body(x_ref, w_ref, b_ref, o4_ref):
    h = jnp.maximum(
        jnp.dot(x_ref[...], w_ref[...],
                preferred_element_type=jnp.float32) + b_ref[...], 0.0)
    for c in range(G):
        o4_ref[c] = h[:, c * GC:(c + 1) * GC]


def _lin0(x, w0t, b0):
    return pl.pallas_call(
        _lin0_body,
        grid=(N // R,),
        in_specs=[pl.BlockSpec((R, F_IN), lambda i: (i, 0)),
                  pl.BlockSpec((F_IN, H), lambda i: (0, 0)),
                  pl.BlockSpec((1, H), lambda i: (0, 0))],
        out_specs=pl.BlockSpec((G, R, GC), lambda i: (0, i, 0)),
        out_shape=jax.ShapeDtypeStruct((G, N, GC), jnp.float32),
    )(x, w0t, b0)


def _combine1_body(a4_ref, h4_ref, w_ref, o4_ref):
    agg = _cat4(a4_ref[...])
    h = _cat4(h4_ref[...])
    out = (1.0 - ALPHA) * agg + ALPHA * h
    xc = jnp.maximum(
        jnp.dot(out, w_ref[...], preferred_element_type=jnp.float32)
        + h, 0.0)
    for c in range(G):
        o4_ref[c] = xc[:, c * GC:(c + 1) * GC]


def _combine1(agg4, h4, wt1):
    return pl.pallas_call(
        _combine1_body,
        grid=(N // R,),
        in_specs=[pl.BlockSpec((G, R, GC), lambda i: (0, i, 0)),
                  pl.BlockSpec((G, R, GC), lambda i: (0, i, 0)),
                  pl.BlockSpec((H, H), lambda i: (0, 0))],
        out_specs=pl.BlockSpec((G, R, GC), lambda i: (0, i, 0)),
        out_shape=jax.ShapeDtypeStruct((G, N, GC), jnp.float32),
    )(agg4, h4, wt1)


def _combine2_body(a4_ref, h4_ref, xc4_ref, w_ref, w1_ref, b1_ref, o_ref):
    agg = _cat4(a4_ref[...])
    h = _cat4(h4_ref[...])
    xc1 = _cat4(xc4_ref[...])
    out = (1.0 - ALPHA) * agg + ALPHA * h
    xc2 = jnp.maximum(
        jnp.dot(out, w_ref[...], preferred_element_type=jnp.float32)
        + xc1, 0.0)
    o_ref[...] = jnp.dot(xc2, w1_ref[...],
                         preferred_element_type=jnp.float32) + b1_ref[...]


def _combine2(agg4, h4, xc14, wt2, w1t, b1):
    return pl.pallas_call(
        _combine2_body,
        grid=(N // R,),
        in_specs=[pl.BlockSpec((G, R, GC), lambda i: (0, i, 0)),
                  pl.BlockSpec((G, R, GC), lambda i: (0, i, 0)),
                  pl.BlockSpec((G, R, GC), lambda i: (0, i, 0)),
                  pl.BlockSpec((H, H), lambda i: (0, 0)),
                  pl.BlockSpec((H, C_OUT), lambda i: (0, 0)),
                  pl.BlockSpec((1, C_OUT), lambda i: (0, 0))],
        out_specs=pl.BlockSpec((R, C_OUT), lambda i: (i, 0)),
        out_shape=jax.ShapeDtypeStruct((N, C_OUT), jnp.float32),
    )(agg4, h4, xc14, wt2, w1t, b1)


def kernel(x, edge_index, lin0_w, lin0_b, lin1_w, lin1_b, conv_w1, conv_w2):
    src2 = edge_index[0].reshape(EK, K)
    dst2 = edge_index[1].reshape(EK, K)
    w0t = lin0_w.T
    b0 = lin0_b.reshape(1, H)
    beta1 = math.log(THETA / 1.0 + 1.0)
    beta2 = math.log(THETA / 2.0 + 1.0)
    eye = jnp.eye(H, dtype=jnp.float32)
    wt1 = (1.0 - beta1) * eye + beta1 * conv_w1
    wt2 = (1.0 - beta2) * eye + beta2 * conv_w2
    w1t = lin1_w.T
    b1 = lin1_b.reshape(1, C_OUT)
    zeros32 = jnp.zeros((ZST, GC), jnp.float32)

    h4 = _lin0(x, w0t, b0)
    agg4 = _sc_scatter(h4, src2, dst2, zeros32)
    xc14 = _combine1(agg4, h4, wt1)
    agg4b = _sc_scatter(xc14, src2, dst2, zeros32)
    return _combine2(agg4b, h4, xc14, wt2, w1t, b1)


# speedup vs baseline: 1.6094x; 1.6094x over previous
import math

import jax
import jax.numpy as jnp
from jax import lax
from jax.experimental import pallas as pl
from jax.experimental.pallas import tpu as pltpu
from jax.experimental.pallas import tpu_sc as plsc

N = 50000
E = 800000
F_IN = 50
H = 128
C_OUT = 121
ALPHA = 0.1
THETA = 0.5

NC = 2
NS = 16
G = 2
GC = H // G
K = 250
EK = E // K
IRT = EK // NS
BRK = 8
NBLK = IRT // BRK
ZST = 3200
ZREM = N - 15 * ZST


def _sc_pass(table, out_g, src2_hbm, dst2_hbm, zeros_hbm,
             dst_blk, src_blk, rows0, rows1, agg,
             gsem0, gsem1, ssem0, ssem1, tid):
    @pl.when(tid < 15)
    def _zero_main():
        pltpu.sync_copy(zeros_hbm, agg.at[pl.ds(tid * ZST, ZST)])

    @pl.when(tid == 15)
    def _zero_rem():
        pltpu.sync_copy(zeros_hbm.at[pl.ds(0, ZREM)],
                        agg.at[pl.ds(15 * ZST, ZREM)])

    plsc.subcore_barrier()

    def blk_body(b, _):
        row_off = tid * IRT + b * BRK
        pltpu.sync_copy(src2_hbm.at[pl.ds(row_off, BRK)], src_blk)
        pltpu.sync_copy(dst2_hbm.at[pl.ds(row_off, BRK)], dst_blk)

        def gather(bb, rows, sem):
            pltpu.async_copy(table.at[src_blk.at[bb]], rows, sem)

        def gwait(rows, sem):
            pltpu.make_async_copy(table.at[src_blk.at[0]], rows, sem).wait()

        def scat(bb, rows, sem):
            pltpu.async_copy(rows, agg.at[dst_blk.at[bb]], sem, add=True)

        def swait(rows, sem):
            pltpu.make_async_copy(rows, agg.at[dst_blk.at[0]], sem).wait()

        gather(0, rows0, gsem0)
        gwait(rows0, gsem0)
        scat(0, rows0, ssem0)
        gather(1, rows1, gsem1)
        gwait(rows1, gsem1)
        scat(1, rows1, ssem1)
        swait(rows0, ssem0)
        gather(2, rows0, gsem0)

        def pair(i, _):
            b0 = 2 * i
            gwait(rows0, gsem0)
            scat(b0, rows0, ssem0)
            swait(rows1, ssem1)
            gather(b0 + 1, rows1, gsem1)
            gwait(rows1, gsem1)
            scat(b0 + 1, rows1, ssem1)
            swait(rows0, ssem0)
            gather(b0 + 2, rows0, gsem0)
            return 0

        lax.fori_loop(1, (BRK - 2) // 2, pair, 0)
        gwait(rows0, gsem0)
        scat(BRK - 2, rows0, ssem0)
        swait(rows1, ssem1)
        gather(BRK - 1, rows1, gsem1)
        gwait(rows1, gsem1)
        scat(BRK - 1, rows1, ssem1)
        swait(rows0, ssem0)
        swait(rows1, ssem1)
        return 0

    lax.fori_loop(0, NBLK, blk_body, 0)
    plsc.subcore_barrier()

    @pl.when(tid < 15)
    def _out_main():
        pltpu.sync_copy(agg.at[pl.ds(tid * ZST, ZST)],
                        out_g.at[pl.ds(tid * ZST, ZST)])

    @pl.when(tid == 15)
    def _out_rem():
        pltpu.sync_copy(agg.at[pl.ds(15 * ZST, ZREM)],
                        out_g.at[pl.ds(15 * ZST, ZREM)])

    plsc.subcore_barrier()


def _sc_scatter_body(h4_hbm, src2_hbm, dst2_hbm, zeros_hbm, out_hbm,
                     dst_blk, src_blk, rows0, rows1, agg,
                     gsem0, gsem1, ssem0, ssem1):
    core = lax.axis_index("c")
    tid = lax.axis_index("s")

    for c in range(NC):
        @pl.when(core == c)
        def _core_work(c=c):
            _sc_pass(h4_hbm.at[c], out_hbm.at[c], src2_hbm, dst2_hbm,
                     zeros_hbm, dst_blk, src_blk, rows0, rows1, agg,
                     gsem0, gsem1, ssem0, ssem1, tid)


def _sc_scatter(h4, src2, dst2, zeros32):
    mesh = plsc.VectorSubcoreMesh(core_axis_name="c", subcore_axis_name="s",
                                  num_cores=NC, num_subcores=NS)
    return pl.kernel(
        _sc_scatter_body,
        out_type=jax.ShapeDtypeStruct((G, N, GC), jnp.bfloat16),
        mesh=mesh,
        scratch_types=[
            pltpu.VMEM((BRK, K), jnp.int32),
            pltpu.VMEM((BRK, K), jnp.int32),
            pltpu.VMEM((K, GC), jnp.bfloat16),
            pltpu.VMEM((K, GC), jnp.bfloat16),
            pltpu.VMEM_SHARED((N, GC), jnp.bfloat16),
            pltpu.SemaphoreType.DMA,
            pltpu.SemaphoreType.DMA,
            pltpu.SemaphoreType.DMA,
            pltpu.SemaphoreType.DMA,
        ],
        compiler_params=pltpu.CompilerParams(use_tc_tiling_on_sc=False),
    )(h4, src2, dst2, zeros32)


R = 1000


def _split4(x):
    return [x[:, c * GC:(c + 1) * GC] for c in range(G)]


def _lin0_body(x_ref, w_ref, b_ref, o_ref, o4_ref):
    h = jnp.maximum(
        jnp.dot(x_ref[...], w_ref[...],
                preferred_element_type=jnp.float32) + b_ref[...], 0.0)
    o_ref[...] = h
    for c in range(G):
        o4_ref[c] = h[:, c * GC:(c + 1) * GC].astype(jnp.bfloat16)


def _lin0(x, w0t, b0):
    return pl.pallas_call(
        _lin0_body,
        grid=(N // R,),
        in_specs=[pl.BlockSpec((R, F_IN), lambda i: (i, 0)),
                  pl.BlockSpec((F_IN, H), lambda i: (0, 0)),
                  pl.BlockSpec((1, H), lambda i: (0, 0))],
        out_specs=[pl.BlockSpec((R, H), lambda i: (i, 0)),
                   pl.BlockSpec((G, R, GC), lambda i: (0, i, 0))],
        out_shape=[jax.ShapeDtypeStruct((N, H), jnp.float32),
                   jax.ShapeDtypeStruct((G, N, GC), jnp.bfloat16)],
    )(x, w0t, b0)


def _combine1_body(a4_ref, h_ref, w_ref, o_ref, o4_ref):
    a4 = a4_ref[...]
    agg = jnp.concatenate([a4[c] for c in range(G)],
                          axis=-1).astype(jnp.float32)
    out = (1.0 - ALPHA) * agg + ALPHA * h_ref[...]
    xc = jnp.maximum(
        jnp.dot(out, w_ref[...], preferred_element_type=jnp.float32)
        + h_ref[...], 0.0)
    o_ref[...] = xc
    for c in range(G):
        o4_ref[c] = xc[:, c * GC:(c + 1) * GC].astype(jnp.bfloat16)


def _combine1(agg4, h, wt1):
    return pl.pallas_call(
        _combine1_body,
        grid=(N // R,),
        in_specs=[pl.BlockSpec((G, R, GC), lambda i: (0, i, 0)),
                  pl.BlockSpec((R, H), lambda i: (i, 0)),
                  pl.BlockSpec((H, H), lambda i: (0, 0))],
        out_specs=[pl.BlockSpec((R, H), lambda i: (i, 0)),
                   pl.BlockSpec((G, R, GC), lambda i: (0, i, 0))],
        out_shape=[jax.ShapeDtypeStruct((N, H), jnp.float32),
                   jax.ShapeDtypeStruct((G, N, GC), jnp.bfloat16)],
    )(agg4, h, wt1)


def _combine2_body(a4_ref, h_ref, xc_ref, w_ref, w1_ref, b1_ref, o_ref):
    a4 = a4_ref[...]
    agg = jnp.concatenate([a4[c] for c in range(G)],
                          axis=-1).astype(jnp.float32)
    out = (1.0 - ALPHA) * agg + ALPHA * h_ref[...]
    xc2 = jnp.maximum(
        jnp.dot(out, w_ref[...], preferred_element_type=jnp.float32)
        + xc_ref[...], 0.0)
    o_ref[...] = jnp.dot(xc2, w1_ref[...],
                         preferred_element_type=jnp.float32) + b1_ref[...]


def _combine2(agg4, h, xc1, wt2, w1t, b1):
    return pl.pallas_call(
        _combine2_body,
        grid=(N // R,),
        in_specs=[pl.BlockSpec((G, R, GC), lambda i: (0, i, 0)),
                  pl.BlockSpec((R, H), lambda i: (i, 0)),
                  pl.BlockSpec((R, H), lambda i: (i, 0)),
                  pl.BlockSpec((H, H), lambda i: (0, 0)),
                  pl.BlockSpec((H, C_OUT), lambda i: (0, 0)),
                  pl.BlockSpec((1, C_OUT), lambda i: (0, 0))],
        out_specs=pl.BlockSpec((R, C_OUT), lambda i: (i, 0)),
        out_shape=jax.ShapeDtypeStruct((N, C_OUT), jnp.float32),
    )(agg4, h, xc1, wt2, w1t, b1)


def kernel(x, edge_index, lin0_w, lin0_b, lin1_w, lin1_b, conv_w1, conv_w2):
    src2 = edge_index[0].reshape(EK, K)
    dst2 = edge_index[1].reshape(EK, K)
    w0t = lin0_w.T
    b0 = lin0_b.reshape(1, H)
    beta1 = math.log(THETA / 1.0 + 1.0)
    beta2 = math.log(THETA / 2.0 + 1.0)
    eye = jnp.eye(H, dtype=jnp.float32)
    wt1 = (1.0 - beta1) * eye + beta1 * conv_w1
    wt2 = (1.0 - beta2) * eye + beta2 * conv_w2
    w1t = lin1_w.T
    b1 = lin1_b.reshape(1, C_OUT)
    zeros32 = jnp.zeros((ZST, GC), jnp.bfloat16)

    h, h4 = _lin0(x, w0t, b0)
    agg4 = _sc_scatter(h4, src2, dst2, zeros32)
    xc1, xc14 = _combine1(agg4, h, wt1)
    agg4b = _sc_scatter(xc14, src2, dst2, zeros32)
    return _combine2(agg4b, h, xc1, wt2, w1t, b1)


# speedup vs baseline: 1.6097x; 1.0002x over previous
import math

import jax
import jax.numpy as jnp
from jax import lax
from jax.experimental import pallas as pl
from jax.experimental.pallas import tpu as pltpu
from jax.experimental.pallas import tpu_sc as plsc

N = 50000
E = 800000
F_IN = 50
H = 128
C_OUT = 121
ALPHA = 0.1
THETA = 0.5

NC = 2
NS = 16
G = 2
GC = H // G
K = 250
EK = E // K
IRT = EK // NS
BRK = 8
NBLK = IRT // BRK
ZST = 3200
ZREM = N - 15 * ZST


def _sc_pass(table, out_g, src2_hbm, dst2_hbm, zeros_hbm,
             dst_blk, src_blk, rows0, rows1, agg,
             gsem0, gsem1, ssem0, ssem1, tid):
    @pl.when(tid < 15)
    def _zero_main():
        pltpu.sync_copy(zeros_hbm, agg.at[pl.ds(tid * ZST, ZST)])

    @pl.when(tid == 15)
    def _zero_rem():
        pltpu.sync_copy(zeros_hbm.at[pl.ds(0, ZREM)],
                        agg.at[pl.ds(15 * ZST, ZREM)])

    plsc.subcore_barrier()

    def blk_body(b, _):
        row_off = tid * IRT + b * BRK
        pltpu.sync_copy(src2_hbm.at[pl.ds(row_off, BRK)], src_blk)
        pltpu.sync_copy(dst2_hbm.at[pl.ds(row_off, BRK)], dst_blk)

        def gather(bb, rows, sem):
            pltpu.async_copy(table.at[src_blk.at[bb]], rows, sem)

        def gwait(rows, sem):
            pltpu.make_async_copy(table.at[src_blk.at[0]], rows, sem).wait()

        def scat(bb, rows, sem):
            pltpu.async_copy(rows, agg.at[dst_blk.at[bb]], sem, add=True)

        def swait(rows, sem):
            pltpu.make_async_copy(rows, agg.at[dst_blk.at[0]], sem).wait()

        gather(0, rows0, gsem0)
        gwait(rows0, gsem0)
        scat(0, rows0, ssem0)
        gather(1, rows1, gsem1)
        gwait(rows1, gsem1)
        scat(1, rows1, ssem1)
        swait(rows0, ssem0)
        gather(2, rows0, gsem0)

        def pair(i, _):
            b0 = 2 * i
            gwait(rows0, gsem0)
            scat(b0, rows0, ssem0)
            swait(rows1, ssem1)
            gather(b0 + 1, rows1, gsem1)
            gwait(rows1, gsem1)
            scat(b0 + 1, rows1, ssem1)
            swait(rows0, ssem0)
            gather(b0 + 2, rows0, gsem0)
            return 0

        lax.fori_loop(1, (BRK - 2) // 2, pair, 0)
        gwait(rows0, gsem0)
        scat(BRK - 2, rows0, ssem0)
        swait(rows1, ssem1)
        gather(BRK - 1, rows1, gsem1)
        gwait(rows1, gsem1)
        scat(BRK - 1, rows1, ssem1)
        swait(rows0, ssem0)
        swait(rows1, ssem1)
        return 0

    lax.fori_loop(0, NBLK, blk_body, 0)
    plsc.subcore_barrier()

    @pl.when(tid < 15)
    def _out_main():
        pltpu.sync_copy(agg.at[pl.ds(tid * ZST, ZST)],
                        out_g.at[pl.ds(tid * ZST, ZST)])

    @pl.when(tid == 15)
    def _out_rem():
        pltpu.sync_copy(agg.at[pl.ds(15 * ZST, ZREM)],
                        out_g.at[pl.ds(15 * ZST, ZREM)])

    plsc.subcore_barrier()


def _sc_scatter_body(h4_hbm, src2_hbm, dst2_hbm, zeros_hbm, out_hbm,
                     dst_blk, src_blk, rows0, rows1, agg,
                     gsem0, gsem1, ssem0, ssem1):
    core = lax.axis_index("c")
    tid = lax.axis_index("s")

    for c in range(NC):
        @pl.when(core == c)
        def _core_work(c=c):
            _sc_pass(h4_hbm.at[c], out_hbm.at[c], src2_hbm, dst2_hbm,
                     zeros_hbm, dst_blk, src_blk, rows0, rows1, agg,
                     gsem0, gsem1, ssem0, ssem1, tid)


def _sc_scatter(h4, src2, dst2, zeros32):
    mesh = plsc.VectorSubcoreMesh(core_axis_name="c", subcore_axis_name="s",
                                  num_cores=NC, num_subcores=NS)
    return pl.kernel(
        _sc_scatter_body,
        out_type=jax.ShapeDtypeStruct((G, N, GC), jnp.bfloat16),
        mesh=mesh,
        scratch_types=[
            pltpu.VMEM((BRK, K), jnp.int32),
            pltpu.VMEM((BRK, K), jnp.int32),
            pltpu.VMEM((K, GC), jnp.bfloat16),
            pltpu.VMEM((K, GC), jnp.bfloat16),
            pltpu.VMEM_SHARED((N, GC), jnp.bfloat16),
            pltpu.SemaphoreType.DMA,
            pltpu.SemaphoreType.DMA,
            pltpu.SemaphoreType.DMA,
            pltpu.SemaphoreType.DMA,
        ],
        compiler_params=pltpu.CompilerParams(use_tc_tiling_on_sc=False),
    )(h4, src2, dst2, zeros32)


R = 1000


def _lin0_body(x_ref, w_ref, b_ref, o_ref, o4_ref):
    h = jnp.maximum(
        jnp.dot(x_ref[...], w_ref[...],
                preferred_element_type=jnp.float32) + b_ref[...], 0.0)
    o_ref[...] = h
    for c in range(G):
        o4_ref[c] = h[:, c * GC:(c + 1) * GC].astype(jnp.bfloat16)


def _lin0(x, w0t, b0):
    return pl.pallas_call(
        _lin0_body,
        grid=(N // R,),
        in_specs=[pl.BlockSpec((R, F_IN), lambda i: (i, 0)),
                  pl.BlockSpec((F_IN, H), lambda i: (0, 0)),
                  pl.BlockSpec((1, H), lambda i: (0, 0))],
        out_specs=[pl.BlockSpec((R, H), lambda i: (i, 0)),
                   pl.BlockSpec((G, R, GC), lambda i: (0, i, 0))],
        out_shape=[jax.ShapeDtypeStruct((N, H), jnp.float32),
                   jax.ShapeDtypeStruct((G, N, GC), jnp.bfloat16)],
    )(x, w0t, b0)


def _combine1_body(a4_ref, h_ref, w_ref, o_ref, o4_ref):
    a4 = a4_ref[...]
    agg = jnp.concatenate([a4[c] for c in range(G)],
                          axis=-1).astype(jnp.float32)
    out = (1.0 - ALPHA) * agg + ALPHA * h_ref[...]
    xc = jnp.maximum(
        jnp.dot(out, w_ref[...], preferred_element_type=jnp.float32)
        + h_ref[...], 0.0)
    o_ref[...] = xc
    for c in range(G):
        o4_ref[c] = xc[:, c * GC:(c + 1) * GC].astype(jnp.bfloat16)


def _combine1(agg4, h, wt1):
    return pl.pallas_call(
        _combine1_body,
        grid=(N // R,),
        in_specs=[pl.BlockSpec((G, R, GC), lambda i: (0, i, 0)),
                  pl.BlockSpec((R, H), lambda i: (i, 0)),
                  pl.BlockSpec((H, H), lambda i: (0, 0))],
        out_specs=[pl.BlockSpec((R, H), lambda i: (i, 0)),
                   pl.BlockSpec((G, R, GC), lambda i: (0, i, 0))],
        out_shape=[jax.ShapeDtypeStruct((N, H), jnp.float32),
                   jax.ShapeDtypeStruct((G, N, GC), jnp.bfloat16)],
    )(agg4, h, wt1)


def _combine2_body(a4_ref, h_ref, xc_ref, w_ref, w1_ref, b1_ref, o_ref):
    a4 = a4_ref[...]
    agg = jnp.concatenate([a4[c] for c in range(G)],
                          axis=-1).astype(jnp.float32)
    out = (1.0 - ALPHA) * agg + ALPHA * h_ref[...]
    xc2 = jnp.maximum(
        jnp.dot(out, w_ref[...], preferred_element_type=jnp.float32)
        + xc_ref[...], 0.0)
    o_ref[...] = jnp.dot(xc2, w1_ref[...],
                         preferred_element_type=jnp.float32) + b1_ref[...]


def _combine2(agg4, h, xc1, wt2, w1t, b1):
    return pl.pallas_call(
        _combine2_body,
        grid=(N // R,),
        in_specs=[pl.BlockSpec((G, R, GC), lambda i: (0, i, 0)),
                  pl.BlockSpec((R, H), lambda i: (i, 0)),
                  pl.BlockSpec((R, H), lambda i: (i, 0)),
                  pl.BlockSpec((H, H), lambda i: (0, 0)),
                  pl.BlockSpec((H, C_OUT), lambda i: (0, 0)),
                  pl.BlockSpec((1, C_OUT), lambda i: (0, 0))],
        out_specs=pl.BlockSpec((R, C_OUT), lambda i: (i, 0)),
        out_shape=jax.ShapeDtypeStruct((N, C_OUT), jnp.float32),
    )(agg4, h, xc1, wt2, w1t, b1)


def kernel(x, edge_index, lin0_w, lin0_b, lin1_w, lin1_b, conv_w1, conv_w2):
    src2 = edge_index[0].reshape(EK, K)
    dst2 = edge_index[1].reshape(EK, K)
    w0t = lin0_w.T
    b0 = lin0_b.reshape(1, H)
    beta1 = math.log(THETA / 1.0 + 1.0)
    beta2 = math.log(THETA / 2.0 + 1.0)
    eye = jnp.eye(H, dtype=jnp.float32)
    wt1 = (1.0 - beta1) * eye + beta1 * conv_w1
    wt2 = (1.0 - beta2) * eye + beta2 * conv_w2
    w1t = lin1_w.T
    b1 = lin1_b.reshape(1, C_OUT)
    zeros32 = jnp.zeros((ZST, GC), jnp.bfloat16)

    h, h4 = _lin0(x, w0t, b0)
    agg4 = _sc_scatter(h4, src2, dst2, zeros32)
    xc1, xc14 = _combine1(agg4, h, wt1)
    agg4b = _sc_scatter(xc14, src2, dst2, zeros32)
    return _combine2(agg4b, h, xc1, wt2, w1t, b1)
